# baseline (speedup 1.0000x reference)
import functools

import jax
import jax.numpy as jnp
from jax import lax
from jax.experimental import pallas as pl
from jax.experimental.pallas import tpu as pltpu
from jax.experimental.pallas import tpu_sc as plsc

B = 4096
D = 128
C = 1000
M_ROWS = 100000

BI = 512
NBLK = B // BI

NC = 2
NS = 16
NW = NC * NS
CH = B // NW


NSTEP = 10
CP_ROWS = M_ROWS // NSTEP


def _mm(i):
  return jnp.minimum(i, NBLK - 1)


def _tc_body(idx_blk_ref, idx_all_ref, x_ref, w_ref, b_ref, mem_ref,
             out_ref, win_ref, newmem_ref):
  newmem_ref[...] = mem_ref[...]

  @pl.when(pl.program_id(0) < NBLK)
  def _compute():
    out_ref[...] = (
        jnp.dot(x_ref[...], w_ref[...], preferred_element_type=jnp.float32)
        + b_ref[...]
    )
    ii = idx_blk_ref[0, 0, :].reshape(BI, 1)
    alljj = idx_all_ref[...]
    eq = ii == alljj
    jio = lax.broadcasted_iota(jnp.int32, (BI, B), 1)
    win = jnp.max(jnp.where(eq, jio, -1), axis=1)
    win_ref[0, :] = idx_blk_ref[0, 0, :]
    win_ref[1, :] = win


_tc_call = pl.pallas_call(
    _tc_body,
    grid=(NSTEP,),
    in_specs=[
        pl.BlockSpec((1, 1, BI), lambda i: (_mm(i), 0, 0)),
        pl.BlockSpec((1, B), lambda i: (0, 0)),
        pl.BlockSpec((BI, D), lambda i: (_mm(i), 0)),
        pl.BlockSpec((D, C), lambda i: (0, 0)),
        pl.BlockSpec((1, C), lambda i: (0, 0)),
        pl.BlockSpec((1, CP_ROWS, D), lambda i: (i, 0, 0)),
    ],
    out_specs=[
        pl.BlockSpec((BI, C), lambda i: (_mm(i), 0)),
        pl.BlockSpec((2, BI), lambda i: (0, _mm(i))),
        pl.BlockSpec((1, CP_ROWS, D), lambda i: (i, 0, 0)),
    ],
    out_shape=[
        jax.ShapeDtypeStruct((B, C), jnp.float32),
        jax.ShapeDtypeStruct((2, B), jnp.int32),
        jax.ShapeDtypeStruct((NSTEP, CP_ROWS, D), jnp.float32),
    ],
)


def _sc_scatter_body(x_hbm, iw_hbm, mem_hbm, iw_v, rows_v, sem):
  wid = lax.axis_index("s") * NC + lax.axis_index("c")
  base = wid * CH
  pltpu.sync_copy(iw_hbm.at[:, pl.ds(base, CH)], iw_v)
  pltpu.async_copy(x_hbm.at[iw_v.at[1]], rows_v, sem).wait()
  pltpu.async_copy(rows_v, mem_hbm.at[iw_v.at[0]], sem).wait()


@functools.cache
def _sc_scatter():
  return functools.partial(
      pl.kernel,
      mesh=plsc.VectorSubcoreMesh(core_axis_name="c", subcore_axis_name="s"),
      scratch_types=[
          pltpu.VMEM((2, CH), jnp.int32),
          pltpu.VMEM((CH, D), jnp.float32),
          pltpu.SemaphoreType.DMA,
      ],
  )(_sc_scatter_body)


def kernel(x, mem, idx, W, b):
  idx32 = idx.astype(jnp.int32)
  out, iw, new_mem3 = _tc_call(
      idx32.reshape(NBLK, 1, BI),
      idx32.reshape(1, B),
      x,
      W,
      b.reshape(1, C),
      mem.reshape(NSTEP, CP_ROWS, D),
  )
  mem_ref = jax.new_ref(new_mem3.reshape(M_ROWS, D))
  _sc_scatter()(x, iw, mem_ref)
  return out, mem_ref[...]


# speedup vs baseline: 1.0008x; 1.0008x over previous
import functools

import jax
import jax.numpy as jnp
from jax import lax
from jax.experimental import pallas as pl
from jax.experimental.pallas import tpu as pltpu
from jax.experimental.pallas import tpu_sc as plsc

B = 4096
D = 128
C = 1000
M_ROWS = 100000

BI = 512
NBLK = B // BI

NC = 2
NS = 16
NW = NC * NS
CH = B // NW


NSTEP = 10
CP_ROWS = M_ROWS // NSTEP


def _mm(i):
  return jnp.minimum(i, NBLK - 1)


def _tc_body(idx_blk_ref, idx_all_ref, x_ref, w_ref, b_ref, mem_ref,
             out_ref, win_ref, newmem_ref):
  newmem_ref[...] = mem_ref[...]

  @pl.when(pl.program_id(0) < NBLK)
  def _compute():
    out_ref[...] = (
        jnp.dot(x_ref[...], w_ref[...], preferred_element_type=jnp.float32)
        + b_ref[...]
    )
    ii = idx_blk_ref[0, 0, :].reshape(BI, 1)
    alljj = idx_all_ref[...]
    eq = ii == alljj
    jio = lax.broadcasted_iota(jnp.int32, (BI, B), 1)
    win = jnp.max(jnp.where(eq, jio, -1), axis=1)
    win_ref[0, 0, :] = win


_tc_call = pl.pallas_call(
    _tc_body,
    grid=(NSTEP,),
    in_specs=[
        pl.BlockSpec((1, 1, BI), lambda i: (_mm(i), 0, 0)),
        pl.BlockSpec((1, B), lambda i: (0, 0)),
        pl.BlockSpec((BI, D), lambda i: (_mm(i), 0)),
        pl.BlockSpec((D, C), lambda i: (0, 0)),
        pl.BlockSpec((1, C), lambda i: (0, 0)),
        pl.BlockSpec((1, CP_ROWS, D), lambda i: (i, 0, 0)),
    ],
    out_specs=[
        pl.BlockSpec((BI, C), lambda i: (_mm(i), 0)),
        pl.BlockSpec((1, 1, BI), lambda i: (_mm(i), 0, 0)),
        pl.BlockSpec((1, CP_ROWS, D), lambda i: (i, 0, 0)),
    ],
    out_shape=[
        jax.ShapeDtypeStruct((B, C), jnp.float32),
        jax.ShapeDtypeStruct((NBLK, 1, BI), jnp.int32),
        jax.ShapeDtypeStruct((NSTEP, CP_ROWS, D), jnp.float32),
    ],
)


CHH = CH // 2


CHH = CH // 2


def _sc_scatter_body(x_hbm, idx_hbm, win_hbm, mem_hbm,
                     idx0, idx1, win0, win1, rows0, rows1,
                     si0, si1, sw0, sw1, sg0, sg1, ss0, ss1):
  wid = lax.axis_index("s") * NC + lax.axis_index("c")
  base = wid * CH
  li0 = pltpu.async_copy(idx_hbm.at[pl.ds(base, CHH)], idx0, si0)
  li1 = pltpu.async_copy(idx_hbm.at[pl.ds(base + CHH, CHH)], idx1, si1)
  lw0 = pltpu.async_copy(win_hbm.at[pl.ds(base, CHH)], win0, sw0)
  lw1 = pltpu.async_copy(win_hbm.at[pl.ds(base + CHH, CHH)], win1, sw1)
  lw0.wait()
  g0 = pltpu.async_copy(x_hbm.at[win0], rows0, sg0)
  lw1.wait()
  g1 = pltpu.async_copy(x_hbm.at[win1], rows1, sg1)
  g0.wait()
  li0.wait()
  s0 = pltpu.async_copy(rows0, mem_hbm.at[idx0], ss0)
  g1.wait()
  li1.wait()
  s1 = pltpu.async_copy(rows1, mem_hbm.at[idx1], ss1)
  s0.wait()
  s1.wait()


@functools.cache
def _sc_scatter():
  return functools.partial(
      pl.kernel,
      mesh=plsc.VectorSubcoreMesh(core_axis_name="c", subcore_axis_name="s"),
      scratch_types=[
          pltpu.VMEM((CHH,), jnp.int32),
          pltpu.VMEM((CHH,), jnp.int32),
          pltpu.VMEM((CHH,), jnp.int32),
          pltpu.VMEM((CHH,), jnp.int32),
          pltpu.VMEM((CHH, D), jnp.float32),
          pltpu.VMEM((CHH, D), jnp.float32),
          pltpu.SemaphoreType.DMA,
          pltpu.SemaphoreType.DMA,
          pltpu.SemaphoreType.DMA,
          pltpu.SemaphoreType.DMA,
          pltpu.SemaphoreType.DMA,
          pltpu.SemaphoreType.DMA,
          pltpu.SemaphoreType.DMA,
          pltpu.SemaphoreType.DMA,
      ],
  )(_sc_scatter_body)


def kernel(x, mem, idx, W, b):
  idx32 = idx.astype(jnp.int32)
  out, win3, new_mem3 = _tc_call(
      idx32.reshape(NBLK, 1, BI),
      idx32.reshape(1, B),
      x,
      W,
      b.reshape(1, C),
      mem.reshape(NSTEP, CP_ROWS, D),
  )
  mem_ref = jax.new_ref(new_mem3.reshape(M_ROWS, D))
  _sc_scatter()(x, idx32, win3.reshape(B), mem_ref)
  return out, mem_ref[...]
